# Initial kernel scaffold; baseline (speedup 1.0000x reference)
import functools

import jax
import jax.numpy as jnp
from jax import lax
from jax.experimental import pallas as pl
from jax.experimental.pallas import tpu as pltpu
from jax.experimental.pallas import tpu_sc as plsc

_N_NODES = 10000
_N_EDGES = 320000
_DIM = 128

_NSUB = 16
_NCORE = 2
_NPAD = 10240
_BINS_PER_SUB = _NPAD // _NSUB
_EDGES_HIST = _N_EDGES // _NSUB
_EDGES_GATHER = _N_EDGES // (_NSUB * _NCORE)


def _sc_weights(target):
    mesh = plsc.VectorSubcoreMesh(core_axis_name="c", subcore_axis_name="s")

    @functools.partial(
        pl.kernel,
        mesh=mesh,
        out_type=jax.ShapeDtypeStruct((_N_EDGES,), jnp.float32),
        scratch_types=[
            pltpu.VMEM((_EDGES_HIST,), jnp.int32),
            pltpu.VMEM((_NPAD,), jnp.float32),
            pltpu.VMEM((_NSUB, _BINS_PER_SUB), jnp.float32),
            pltpu.VMEM((_NPAD,), jnp.float32),
            pltpu.VMEM((_EDGES_GATHER,), jnp.int32),
            pltpu.VMEM((_EDGES_GATHER,), jnp.float32),
            pltpu.VMEM_SHARED((_NSUB, _NSUB, _BINS_PER_SUB), jnp.float32),
            pltpu.VMEM_SHARED((_NPAD,), jnp.float32),
        ],
    )
    def body(target_hbm, w_hbm, tv, hist, comb, wtab, tg, wout, sh_hist, sh_w):
        s = lax.axis_index("s")
        c = lax.axis_index("c")

        pltpu.sync_copy(target_hbm.at[pl.ds(s * _EDGES_HIST, _EDGES_HIST)], tv)
        zeros16 = jnp.zeros((16,), jnp.float32)

        def zero_body(j, carry):
            hist[pl.ds(j * 16, 16)] = zeros16
            return carry

        lax.fori_loop(0, _NPAD // 16, zero_body, 0)

        ones16 = jnp.ones((16,), jnp.float32)

        def hist_body(i, carry):
            idx = tv[pl.ds(i * 16, 16)]
            plsc.addupdate_scatter(hist, [idx], ones16)
            return carry

        lax.fori_loop(0, _EDGES_HIST // 16, hist_body, 0)

        for p in range(_NSUB):
            pltpu.sync_copy(
                hist.at[pl.ds(p * _BINS_PER_SUB, _BINS_PER_SUB)],
                sh_hist.at[p, s],
            )
        plsc.subcore_barrier()

        pltpu.sync_copy(sh_hist.at[s], comb)

        def comb_body(j, carry):
            acc = comb[0, pl.ds(j * 16, 16)]
            for r in range(1, _NSUB):
                acc = acc + comb[r, pl.ds(j * 16, 16)]
            wtab[pl.ds(j * 16, 16)] = 1.0 / acc
            return carry

        lax.fori_loop(0, _BINS_PER_SUB // 16, comb_body, 0)
        pltpu.sync_copy(
            wtab.at[pl.ds(0, _BINS_PER_SUB)],
            sh_w.at[pl.ds(s * _BINS_PER_SUB, _BINS_PER_SUB)],
        )
        plsc.subcore_barrier()

        pltpu.sync_copy(sh_w, wtab)
        wid = s * _NCORE + c
        base = wid * _EDGES_GATHER
        pltpu.sync_copy(target_hbm.at[pl.ds(base, _EDGES_GATHER)], tg)

        def gather_body(i, carry):
            idx = tg[pl.ds(i * 16, 16)]
            wout[pl.ds(i * 16, 16)] = plsc.load_gather(wtab, [idx])
            return carry

        lax.fori_loop(0, _EDGES_GATHER // 16, gather_body, 0)
        pltpu.sync_copy(wout, w_hbm.at[pl.ds(base, _EDGES_GATHER)])

    return body(target)


_ROWS_PER_BLOCK = 512
_WROWS = _ROWS_PER_BLOCK // 128


def _scale_body(m_ref, w_ref, o_ref):
    ones = jnp.ones((1, 128), jnp.float32)
    w = w_ref[...]
    for g in range(_WROWS):
        bcast = lax.dot_general(
            w[g : g + 1, :],
            ones,
            (((0,), (0,)), ((), ())),
            preferred_element_type=jnp.float32,
        )
        rows = pl.ds(g * 128, 128)
        o_ref[rows, :] = m_ref[rows, :] * bcast


def kernel(source, target, message):
    del source
    target = target.astype(jnp.int32)
    weight = _sc_weights(target)
    w2d = weight.reshape(_N_EDGES // 128, 128)
    grid = (_N_EDGES // _ROWS_PER_BLOCK,)
    return pl.pallas_call(
        _scale_body,
        grid=grid,
        in_specs=[
            pl.BlockSpec((_ROWS_PER_BLOCK, _DIM), lambda i: (i, 0)),
            pl.BlockSpec((_WROWS, 128), lambda i: (i, 0)),
        ],
        out_specs=pl.BlockSpec((_ROWS_PER_BLOCK, _DIM), lambda i: (i, 0)),
        out_shape=jax.ShapeDtypeStruct((_N_EDGES, _DIM), jnp.float32),
    )(message, w2d)


# speedup vs baseline: 3.1656x; 3.1656x over previous
import functools

import jax
import jax.numpy as jnp
from jax import lax
from jax.experimental import pallas as pl
from jax.experimental.pallas import tpu as pltpu
from jax.experimental.pallas import tpu_sc as plsc

_N_NODES = 10000
_N_EDGES = 320000
_DIM = 128

_NSUB = 16
_NCORE = 2
_NPAD = 10240
_STRIPE = _NPAD // _NSUB
_EHIST = _N_EDGES // _NSUB
_EGATH = _N_EDGES // (_NSUB * _NCORE)


def _sc_weights(target):
    mesh = plsc.VectorSubcoreMesh(core_axis_name="c", subcore_axis_name="s")

    @functools.partial(
        pl.kernel,
        mesh=mesh,
        out_type=jax.ShapeDtypeStruct((_N_EDGES,), jnp.float32),
        scratch_types=[
            pltpu.VMEM((_EHIST,), jnp.int32),
            pltpu.VMEM((_EHIST,), jnp.float32),
            pltpu.VMEM((_STRIPE,), jnp.float32),
            pltpu.VMEM((_EGATH,), jnp.int32),
            pltpu.VMEM((_EGATH,), jnp.float32),
            pltpu.VMEM_SHARED((_NPAD,), jnp.float32),
            pltpu.SemaphoreType.DMA,
        ],
    )
    def body(target_hbm, w_hbm, tv, ones, stripe, tg, wout, sh_hist, sem):
        s = lax.axis_index("s")
        c = lax.axis_index("c")

        pltpu.sync_copy(target_hbm.at[pl.ds(s * _EHIST, _EHIST)], tv)
        wid = s * _NCORE + c
        base = wid * _EGATH
        pltpu.sync_copy(target_hbm.at[pl.ds(base, _EGATH)], tg)
        ones16 = jnp.ones((16,), jnp.float32)
        zeros16 = jnp.zeros((16,), jnp.float32)

        def fill_body(i, carry):
            ones[pl.ds(i * 16, 16)] = ones16
            return carry

        lax.fori_loop(0, _EHIST // 16, fill_body, 0)

        def zero_body(i, carry):
            stripe[pl.ds(i * 16, 16)] = zeros16
            return carry

        lax.fori_loop(0, _STRIPE // 16, zero_body, 0)

        pltpu.sync_copy(stripe, sh_hist.at[pl.ds(s * _STRIPE, _STRIPE)])
        plsc.subcore_barrier()

        pltpu.sync_copy(ones, sh_hist.at[tv], add=True)
        plsc.subcore_barrier()

        pltpu.sync_copy(sh_hist.at[pl.ds(s * _STRIPE, _STRIPE)], stripe)

        def inv_body(i, carry):
            lanes = pl.ds(i * 16, 16)
            stripe[lanes] = 1.0 / stripe[lanes]
            return carry

        lax.fori_loop(0, _STRIPE // 16, inv_body, 0)
        pltpu.sync_copy(stripe, sh_hist.at[pl.ds(s * _STRIPE, _STRIPE)])
        plsc.subcore_barrier()

        pltpu.async_copy(sh_hist.at[tg], wout, sem).wait()
        pltpu.sync_copy(wout, w_hbm.at[pl.ds(base, _EGATH)])

    return body(target)


_ROWS_PER_BLOCK = 512
_WROWS = _ROWS_PER_BLOCK // 128


def _scale_body(m_ref, w_ref, o_ref):
    ones = jnp.ones((1, 128), jnp.float32)
    w = w_ref[0]
    for g in range(_WROWS):
        bcast = lax.dot_general(
            w[g : g + 1, :],
            ones,
            (((0,), (0,)), ((), ())),
            preferred_element_type=jnp.float32,
        )
        rows = pl.ds(g * 128, 128)
        o_ref[rows, :] = m_ref[rows, :] * bcast


def kernel(source, target, message):
    del source
    weight = _sc_weights(target.astype(jnp.int32))
    w3d = weight.reshape(_N_EDGES // _ROWS_PER_BLOCK, _WROWS, 128)
    grid = (_N_EDGES // _ROWS_PER_BLOCK,)
    return pl.pallas_call(
        _scale_body,
        grid=grid,
        in_specs=[
            pl.BlockSpec((_ROWS_PER_BLOCK, _DIM), lambda i: (i, 0)),
            pl.BlockSpec((1, _WROWS, 128), lambda i: (i, 0, 0)),
        ],
        out_specs=pl.BlockSpec((_ROWS_PER_BLOCK, _DIM), lambda i: (i, 0)),
        out_shape=jax.ShapeDtypeStruct((_N_EDGES, _DIM), jnp.float32),
    )(message, w3d)


# speedup vs baseline: 4.5793x; 1.4466x over previous
import functools

import jax
import jax.numpy as jnp
from jax import lax
from jax.experimental import pallas as pl
from jax.experimental.pallas import tpu as pltpu
from jax.experimental.pallas import tpu_sc as plsc

_N_NODES = 10000
_N_EDGES = 320000
_DIM = 128

_NSUB = 16
_NCORE = 2
_NPAD = 10240
_STRIPE = _NPAD // _NSUB
_EHIST = _N_EDGES // _NSUB
_EGATH = _N_EDGES // (_NSUB * _NCORE)


def _sc_weights(target):
    mesh = plsc.VectorSubcoreMesh(core_axis_name="c", subcore_axis_name="s")

    @functools.partial(
        pl.kernel,
        mesh=mesh,
        out_type=jax.ShapeDtypeStruct((_N_EDGES,), jnp.float32),
        scratch_types=[
            pltpu.VMEM((_EHIST,), jnp.int32),
            pltpu.VMEM((_EHIST,), jnp.float32),
            pltpu.VMEM((_STRIPE,), jnp.float32),
            pltpu.VMEM((_EGATH,), jnp.int32),
            pltpu.VMEM((_EGATH,), jnp.float32),
            pltpu.VMEM_SHARED((_NPAD,), jnp.float32),
            pltpu.SemaphoreType.DMA,
        ],
    )
    def body(target_hbm, w_hbm, tv, ones, stripe, tg, wout, sh_hist, sem):
        s = lax.axis_index("s")
        c = lax.axis_index("c")

        pltpu.sync_copy(target_hbm.at[pl.ds(s * _EHIST, _EHIST)], tv)
        wid = s * _NCORE + c
        base = wid * _EGATH
        pltpu.sync_copy(target_hbm.at[pl.ds(base, _EGATH)], tg)
        ones16 = jnp.ones((16,), jnp.float32)
        zeros16 = jnp.zeros((16,), jnp.float32)

        def fill_body(i, carry):
            ones[pl.ds(i * 16, 16)] = ones16
            return carry

        lax.fori_loop(0, _EHIST // 16, fill_body, 0)

        def zero_body(i, carry):
            stripe[pl.ds(i * 16, 16)] = zeros16
            return carry

        lax.fori_loop(0, _STRIPE // 16, zero_body, 0)

        pltpu.sync_copy(stripe, sh_hist.at[pl.ds(s * _STRIPE, _STRIPE)])
        plsc.subcore_barrier()

        pltpu.sync_copy(ones, sh_hist.at[tv], add=True)
        plsc.subcore_barrier()

        pltpu.sync_copy(sh_hist.at[pl.ds(s * _STRIPE, _STRIPE)], stripe)

        def inv_body(i, carry):
            lanes = pl.ds(i * 16, 16)
            stripe[lanes] = 1.0 / stripe[lanes]
            return carry

        lax.fori_loop(0, _STRIPE // 16, inv_body, 0)
        pltpu.sync_copy(stripe, sh_hist.at[pl.ds(s * _STRIPE, _STRIPE)])
        plsc.subcore_barrier()

        pltpu.async_copy(sh_hist.at[tg], wout, sem).wait()
        pltpu.sync_copy(wout, w_hbm.at[pl.ds(base, _EGATH)])

    return body(target)


_ROWS_PER_BLOCK = 2560


def _scale_body(m_ref, w_ref, o_ref):
    o_ref[...] = m_ref[...] * w_ref[...]


def kernel(source, target, message):
    del source
    weight = _sc_weights(target.astype(jnp.int32))
    wcol = weight.reshape(_N_EDGES, 1)
    grid = (_N_EDGES // _ROWS_PER_BLOCK,)
    return pl.pallas_call(
        _scale_body,
        grid=grid,
        in_specs=[
            pl.BlockSpec((_ROWS_PER_BLOCK, _DIM), lambda i: (i, 0)),
            pl.BlockSpec((_ROWS_PER_BLOCK, 1), lambda i: (i, 0)),
        ],
        out_specs=pl.BlockSpec((_ROWS_PER_BLOCK, _DIM), lambda i: (i, 0)),
        out_shape=jax.ShapeDtypeStruct((_N_EDGES, _DIM), jnp.float32),
    )(message, wcol)


# speedup vs baseline: 4.9806x; 1.0877x over previous
import functools

import jax
import jax.numpy as jnp
from jax import lax
from jax.experimental import pallas as pl
from jax.experimental.pallas import tpu as pltpu
from jax.experimental.pallas import tpu_sc as plsc

_N_NODES = 10000
_N_EDGES = 320000
_DIM = 128

_NSUB = 16
_NCORE = 2
_NPAD = 10240
_STRIPE = _NPAD // _NSUB
_EHIST = _N_EDGES // _NSUB
_EGATH = _N_EDGES // (_NSUB * _NCORE)


def _sc_weights(target):
    mesh = plsc.VectorSubcoreMesh(core_axis_name="c", subcore_axis_name="s")

    @functools.partial(
        pl.kernel,
        mesh=mesh,
        out_type=jax.ShapeDtypeStruct((_N_EDGES,), jnp.float32),
        scratch_types=[
            pltpu.VMEM((_EHIST,), jnp.int32),
            pltpu.VMEM((_EHIST,), jnp.float32),
            pltpu.VMEM((_STRIPE,), jnp.float32),
            pltpu.VMEM((_EGATH,), jnp.int32),
            pltpu.VMEM((_EGATH,), jnp.float32),
            pltpu.VMEM_SHARED((_NPAD,), jnp.float32),
            pltpu.SemaphoreType.DMA,
        ],
    )
    def body(target_hbm, w_hbm, tv, ones, stripe, tg, wout, sh_hist, sem):
        s = lax.axis_index("s")
        c = lax.axis_index("c")

        pltpu.sync_copy(target_hbm.at[pl.ds(s * _EHIST, _EHIST)], tv)
        wid = s * _NCORE + c
        base = wid * _EGATH
        pltpu.sync_copy(target_hbm.at[pl.ds(base, _EGATH)], tg)
        ones16 = jnp.ones((16,), jnp.float32)
        zeros16 = jnp.zeros((16,), jnp.float32)

        def fill_body(i, carry):
            ones[pl.ds(i * 16, 16)] = ones16
            return carry

        lax.fori_loop(0, _EHIST // 16, fill_body, 0)

        def zero_body(i, carry):
            stripe[pl.ds(i * 16, 16)] = zeros16
            return carry

        lax.fori_loop(0, _STRIPE // 16, zero_body, 0)

        pltpu.sync_copy(stripe, sh_hist.at[pl.ds(s * _STRIPE, _STRIPE)])
        plsc.subcore_barrier()

        pltpu.sync_copy(ones, sh_hist.at[tv], add=True)
        plsc.subcore_barrier()

        pltpu.sync_copy(sh_hist.at[pl.ds(s * _STRIPE, _STRIPE)], stripe)

        def inv_body(i, carry):
            lanes = pl.ds(i * 16, 16)
            stripe[lanes] = 1.0 / stripe[lanes]
            return carry

        lax.fori_loop(0, _STRIPE // 16, inv_body, 0)
        pltpu.sync_copy(stripe, sh_hist.at[pl.ds(s * _STRIPE, _STRIPE)])
        plsc.subcore_barrier()

        pltpu.async_copy(sh_hist.at[tg], wout, sem).wait()
        pltpu.sync_copy(wout, w_hbm.at[pl.ds(base, _EGATH)])

    return body(target)


_ROWS_PER_BLOCK = 6400


def _scale_body(m_ref, w_ref, o_ref):
    o_ref[...] = m_ref[...] * w_ref[...]


def kernel(source, target, message):
    del source
    weight = _sc_weights(target.astype(jnp.int32))
    wcol = weight.reshape(_N_EDGES, 1)
    grid = (_N_EDGES // _ROWS_PER_BLOCK,)
    return pl.pallas_call(
        _scale_body,
        grid=grid,
        in_specs=[
            pl.BlockSpec((_ROWS_PER_BLOCK, _DIM), lambda i: (i, 0)),
            pl.BlockSpec((_ROWS_PER_BLOCK, 1), lambda i: (i, 0)),
        ],
        out_specs=pl.BlockSpec((_ROWS_PER_BLOCK, _DIM), lambda i: (i, 0)),
        out_shape=jax.ShapeDtypeStruct((_N_EDGES, _DIM), jnp.float32),
    )(message, wcol)


# speedup vs baseline: 5.0347x; 1.0109x over previous
import functools

import jax
import jax.numpy as jnp
from jax import lax
from jax.experimental import pallas as pl
from jax.experimental.pallas import tpu as pltpu
from jax.experimental.pallas import tpu_sc as plsc

_N_NODES = 10000
_N_EDGES = 320000
_DIM = 128

_NSUB = 16
_NCORE = 2
_NPAD = 10240
_STRIPE = _NPAD // _NSUB
_EHIST = _N_EDGES // _NSUB
_EGATH = _N_EDGES // (_NSUB * _NCORE)


def _sc_weights(target):
    mesh = plsc.VectorSubcoreMesh(core_axis_name="c", subcore_axis_name="s")

    @functools.partial(
        pl.kernel,
        mesh=mesh,
        out_type=jax.ShapeDtypeStruct((_N_EDGES,), jnp.float32),
        scratch_types=[
            pltpu.VMEM((_EHIST,), jnp.int32),
            pltpu.VMEM((_EHIST,), jnp.float32),
            pltpu.VMEM((_STRIPE,), jnp.float32),
            pltpu.VMEM((_EGATH,), jnp.int32),
            pltpu.VMEM((_EGATH,), jnp.float32),
            pltpu.VMEM_SHARED((_NPAD,), jnp.float32),
            pltpu.SemaphoreType.DMA,
        ],
    )
    def body(target_hbm, ones_hbm, w_hbm, tv, ones, stripe, tg, wout, sh_hist, sem):
        s = lax.axis_index("s")
        c = lax.axis_index("c")

        pltpu.sync_copy(target_hbm.at[pl.ds(s * _EHIST, _EHIST)], tv)
        wid = s * _NCORE + c
        base = wid * _EGATH
        pltpu.sync_copy(target_hbm.at[pl.ds(base, _EGATH)], tg)
        pltpu.sync_copy(ones_hbm, ones)
        zeros16 = jnp.zeros((16,), jnp.float32)

        def zero_body(i, carry):
            stripe[pl.ds(i * 16, 16)] = zeros16
            return carry

        lax.fori_loop(0, _STRIPE // 16, zero_body, 0)

        pltpu.sync_copy(stripe, sh_hist.at[pl.ds(s * _STRIPE, _STRIPE)])
        plsc.subcore_barrier()

        pltpu.sync_copy(ones, sh_hist.at[tv], add=True)
        plsc.subcore_barrier()

        pltpu.sync_copy(sh_hist.at[pl.ds(s * _STRIPE, _STRIPE)], stripe)

        def inv_body(i, carry):
            lanes = pl.ds(i * 16, 16)
            stripe[lanes] = 1.0 / stripe[lanes]
            return carry

        lax.fori_loop(0, _STRIPE // 16, inv_body, 0)
        pltpu.sync_copy(stripe, sh_hist.at[pl.ds(s * _STRIPE, _STRIPE)])
        plsc.subcore_barrier()

        pltpu.async_copy(sh_hist.at[tg], wout, sem).wait()
        pltpu.sync_copy(wout, w_hbm.at[pl.ds(base, _EGATH)])

    return body(target, jnp.ones((_EHIST,), jnp.float32))


_ROWS_PER_BLOCK = 6400


def _scale_body(m_ref, w_ref, o_ref):
    o_ref[...] = m_ref[...] * w_ref[...]


def kernel(source, target, message):
    del source
    weight = _sc_weights(target.astype(jnp.int32))
    wcol = weight.reshape(_N_EDGES, 1)
    grid = (_N_EDGES // _ROWS_PER_BLOCK,)
    return pl.pallas_call(
        _scale_body,
        grid=grid,
        in_specs=[
            pl.BlockSpec((_ROWS_PER_BLOCK, _DIM), lambda i: (i, 0)),
            pl.BlockSpec((_ROWS_PER_BLOCK, 1), lambda i: (i, 0)),
        ],
        out_specs=pl.BlockSpec((_ROWS_PER_BLOCK, _DIM), lambda i: (i, 0)),
        out_shape=jax.ShapeDtypeStruct((_N_EDGES, _DIM), jnp.float32),
        compiler_params=pltpu.CompilerParams(
            dimension_semantics=("parallel",),
        ),
    )(message, wcol)


# speedup vs baseline: 10.1835x; 2.0226x over previous
import functools

import jax
import jax.numpy as jnp
from jax import lax
from jax.experimental import pallas as pl
from jax.experimental.pallas import tpu as pltpu
from jax.experimental.pallas import tpu_sc as plsc

_N_NODES = 10000
_N_EDGES = 320000
_DIM = 128

_NSUB = 16
_NCORE = 2
_NPAD = 10240
_STRIPE = _NPAD // _NSUB
_EHIST = _N_EDGES // _NSUB
_EGATH = _N_EDGES // (_NSUB * _NCORE)


def _sc_weights(target):
    mesh = plsc.VectorSubcoreMesh(core_axis_name="c", subcore_axis_name="s")

    @functools.partial(
        pl.kernel,
        mesh=mesh,
        out_type=jax.ShapeDtypeStruct((_N_EDGES,), jnp.float32),
        scratch_types=[
            pltpu.VMEM((_EHIST,), jnp.int32),
            pltpu.VMEM((_EHIST,), jnp.float32),
            pltpu.VMEM((_STRIPE,), jnp.float32),
            pltpu.VMEM((_EGATH,), jnp.float32),
            pltpu.VMEM_SHARED((_NPAD,), jnp.float32),
            pltpu.SemaphoreType.DMA,
        ],
    )
    def body(target_hbm, ones_hbm, w_hbm, tv, ones, stripe, wout, sh_hist, sem):
        s = lax.axis_index("s")
        c = lax.axis_index("c")

        pltpu.sync_copy(target_hbm.at[pl.ds(s * _EHIST, _EHIST)], tv)
        pltpu.sync_copy(ones_hbm, ones)
        zeros16 = jnp.zeros((16,), jnp.float32)

        def zero_body(i, carry):
            stripe[pl.ds(i * 16, 16)] = zeros16
            return carry

        lax.fori_loop(0, _STRIPE // 16, zero_body, 0)

        pltpu.sync_copy(stripe, sh_hist.at[pl.ds(s * _STRIPE, _STRIPE)])
        plsc.subcore_barrier()

        pltpu.sync_copy(ones, sh_hist.at[tv], add=True)
        plsc.subcore_barrier()

        pltpu.sync_copy(sh_hist.at[pl.ds(s * _STRIPE, _STRIPE)], stripe)

        def inv_body(i, carry):
            lanes = pl.ds(i * 16, 16)
            stripe[lanes] = 1.0 / stripe[lanes]
            return carry

        lax.fori_loop(0, _STRIPE // 16, inv_body, 0)
        pltpu.sync_copy(stripe, sh_hist.at[pl.ds(s * _STRIPE, _STRIPE)])
        plsc.subcore_barrier()

        base = s * _EHIST + c * _EGATH
        pltpu.async_copy(
            sh_hist.at[tv.at[pl.ds(c * _EGATH, _EGATH)]], wout, sem
        ).wait()
        pltpu.sync_copy(wout, w_hbm.at[pl.ds(base, _EGATH)])

    return body(target, jnp.ones((_EHIST,), jnp.float32))


_ROWS_PER_BLOCK = 6400
_WR = _ROWS_PER_BLOCK // 128
_NBLK = _N_EDGES // _ROWS_PER_BLOCK


def _scale_body(m_ref, w_ref, o_ref):
    ones = jnp.ones((1, 128), jnp.float32)
    w = w_ref[0]
    for g in range(_WR):
        bcast = lax.dot_general(
            w[g : g + 1, :],
            ones,
            (((0,), (0,)), ((), ())),
            preferred_element_type=jnp.float32,
        )
        rows = pl.ds(g * 128, 128)
        o_ref[rows, :] = m_ref[rows, :] * bcast


def kernel(source, target, message):
    del source
    weight = _sc_weights(target.astype(jnp.int32))
    w3d = weight.reshape(_NBLK, _WR, 128)
    return pl.pallas_call(
        _scale_body,
        grid=(_NBLK,),
        in_specs=[
            pl.BlockSpec((_ROWS_PER_BLOCK, _DIM), lambda i: (i, 0)),
            pl.BlockSpec((1, _WR, 128), lambda i: (i, 0, 0)),
        ],
        out_specs=pl.BlockSpec((_ROWS_PER_BLOCK, _DIM), lambda i: (i, 0)),
        out_shape=jax.ShapeDtypeStruct((_N_EDGES, _DIM), jnp.float32),
    )(message, w3d)
